# Initial kernel scaffold; baseline (speedup 1.0000x reference)
import jax
import jax.numpy as jnp
from jax.experimental import pallas as pl


def kernel(g_emb, w_emb, soft_emb, W_ih, W_hh, b_ih, b_hh):
    raise NotImplementedError("write your pallas kernel here")


# speedup vs baseline: 1.5113x; 1.5113x over previous
import jax
import jax.numpy as jnp
import numpy as np
from jax.experimental import pallas as pl
from jax.experimental.pallas import tpu as pltpu

_STEPS = 24
_A = 8
_H = 1024
_F32 = jnp.float32
_HI = jax.lax.Precision.HIGHEST


def _gumbel_table():
    skey = jax.random.key(42)
    rows = [
        jax.random.gumbel(jax.random.fold_in(skey, s), (1, _A), _F32)
        for s in range(_STEPS)
    ]
    return jnp.concatenate(rows, axis=0)


def _prep_body(w0_ref, rows_ref, e_ref):
    rows_t = rows_ref[...].T
    et = jax.lax.dot_general(
        w0_ref[...], rows_t, (((1,), (0,)), ((), ())),
        preferred_element_type=_F32, precision=_HI)
    e_ref[...] = et.T


def _main_body(e_ref, whh0_ref, w1i_ref, w1h_ref, b_ref, soft_ref, gum_ref,
               stats_ref, arch_ref):
    H = _H
    iota_a = jax.lax.broadcasted_iota(jnp.int32, (1, _A), 1)
    iota_t = jax.lax.broadcasted_iota(jnp.int32, (1, _STEPS), 1)
    b0 = b_ref[0:1, :]
    b1 = b_ref[1:2, :]

    def cell(gates, c):
        i_g = gates[:, 0:H]
        f_g = gates[:, H:2 * H]
        g_g = gates[:, 2 * H:3 * H]
        o_g = gates[:, 3 * H:4 * H]
        c_new = jax.nn.sigmoid(f_g) * c + jax.nn.sigmoid(i_g) * jnp.tanh(g_g)
        h_new = jax.nn.sigmoid(o_g) * jnp.tanh(c_new)
        return h_new, c_new

    def mv(x, w_ref):
        return jax.lax.dot_general(
            x, w_ref[...], (((1,), (0,)), ((), ())),
            preferred_element_type=_F32, precision=_HI)

    def step_fn(t, carry):
        x0e, h0, c0, h1, c1, lp_row, ent_row, act_row = carry
        g0 = x0e + mv(h0, whh0_ref) + b0
        h0n, c0n = cell(g0, c0)
        g1 = mv(h0n, w1i_ref) + mv(h1, w1h_ref) + b1
        h1n, c1n = cell(g1, c1)
        logits = mv(h1n, soft_ref)
        m = jnp.max(logits)
        logp = logits - (m + jnp.log(jnp.sum(jnp.exp(logits - m))))
        z = logits + gum_ref[pl.ds(t, 1), :]
        a = jnp.min(jnp.where(z >= jnp.max(z), iota_a, _A)).astype(jnp.int32)
        onehot = iota_a == a
        lp = jnp.sum(jnp.where(onehot, logp, 0.0))
        ent = -jnp.sum(jnp.exp(logp) * logp)
        oh9 = (jax.lax.broadcasted_iota(jnp.int32, (1, 9), 1) == a + 1)
        x0e_next = mv(oh9.astype(_F32), e_ref)
        tmask = iota_t == t
        lp_row = jnp.where(tmask, lp, lp_row)
        ent_row = jnp.where(tmask, ent, ent_row)
        act_row = jnp.where(tmask, a, act_row)
        return (x0e_next, h0n, c0n, h1n, c1n, lp_row, ent_row, act_row)

    zvec = jnp.zeros((1, H), _F32)
    init = (e_ref[0:1, :], zvec, zvec, zvec, zvec,
            jnp.zeros((1, _STEPS), _F32), jnp.zeros((1, _STEPS), _F32),
            jnp.zeros((1, _STEPS), jnp.int32))
    carry = jax.lax.fori_loop(0, _STEPS, step_fn, init)
    _, _, _, _, _, lp_row, ent_row, act_row = carry
    stats_ref[0:1, :] = lp_row
    stats_ref[1:2, :] = ent_row
    arch_ref[...] = act_row


def kernel(g_emb, w_emb, soft_emb, W_ih, W_hh, b_ih, b_hh):
    rows = jnp.concatenate([g_emb, w_emb], axis=0)
    e = pl.pallas_call(
        _prep_body,
        out_shape=jax.ShapeDtypeStruct((9, 4096), _F32),
        compiler_params=pltpu.CompilerParams(
            vmem_limit_bytes=64 * 1024 * 1024),
    )(W_ih[0], rows)

    whh0 = W_hh[0].T
    w1i = W_ih[1].T
    w1h = W_hh[1].T
    b = b_ih + b_hh
    gum = _gumbel_table()

    stats, arch_row = pl.pallas_call(
        _main_body,
        out_shape=[
            jax.ShapeDtypeStruct((2, _STEPS), _F32),
            jax.ShapeDtypeStruct((1, _STEPS), jnp.int32),
        ],
        compiler_params=pltpu.CompilerParams(
            vmem_limit_bytes=100 * 1024 * 1024),
    )(e, whh0, w1i, w1h, b, soft_emb, gum)
    return stats, arch_row[0]


# speedup vs baseline: 2.4527x; 1.6229x over previous
import jax
import jax.numpy as jnp
import numpy as np
from jax.experimental import pallas as pl
from jax.experimental.pallas import tpu as pltpu

_STEPS = 24
_A = 8
_H = 1024
_F32 = jnp.float32
_BF16 = jnp.bfloat16
_HI = jax.lax.Precision.HIGHEST


def _gumbel_table():
    skey = jax.random.key(42)
    rows = [
        jax.random.gumbel(jax.random.fold_in(skey, s), (1, _A), _F32)
        for s in range(_STEPS)
    ]
    return jnp.concatenate(rows, axis=0)


def _hilo(w):
    hi = w.astype(_BF16)
    lo = (w - hi.astype(_F32)).astype(_BF16)
    return hi, lo


def _prep_body(w0_ref, rows_ref, e_ref):
    rows_t = rows_ref[...].T
    et = jax.lax.dot_general(
        w0_ref[...], rows_t, (((1,), (0,)), ((), ())),
        preferred_element_type=_F32, precision=_HI)
    e_ref[...] = et.T


def _main_body(e_ref, w0_ref, w1_ref, b_ref, soft_ref, gum_ref,
               stats_ref, arch_ref):
    H = _H
    iota_a = jax.lax.broadcasted_iota(jnp.int32, (1, _A), 1)
    iota_t = jax.lax.broadcasted_iota(jnp.int32, (1, _STEPS), 1)
    b0 = b_ref[0:1, :]
    b1 = b_ref[1:2, :]
    zrow = jnp.zeros((1, H), _BF16)

    def cell(gates, c):
        i_g = gates[:, 0:H]
        f_g = gates[:, H:2 * H]
        g_g = gates[:, 2 * H:3 * H]
        o_g = gates[:, 3 * H:4 * H]
        c_new = jax.nn.sigmoid(f_g) * c + jax.nn.sigmoid(i_g) * jnp.tanh(g_g)
        h_new = jax.nn.sigmoid(o_g) * jnp.tanh(c_new)
        return h_new, c_new

    def bdot(act, w_ref):
        r = jax.lax.dot_general(
            act, w_ref[...], (((1,), (0,)), ((), ())),
            preferred_element_type=_F32)
        return r[0:1, :] + r[1:2, :]

    def step_fn(t, carry):
        x0e, h0, c0, h1, c1, lp_row, ent_row, act_row = carry
        h0h, h0l = _hilo(h0)
        a0 = jnp.concatenate([
            jnp.concatenate([h0h, h0h], axis=1),
            jnp.concatenate([h0l, zrow], axis=1)], axis=0)
        g0 = x0e + bdot(a0, w0_ref) + b0
        h0n, c0n = cell(g0, c0)
        xh, xl = _hilo(h0n)
        hh, hl = _hilo(h1)
        a1 = jnp.concatenate([
            jnp.concatenate([xh, xh, hh, hh], axis=1),
            jnp.concatenate([xl, zrow, hl, zrow], axis=1)], axis=0)
        g1 = bdot(a1, w1_ref) + b1
        h1n, c1n = cell(g1, c1)
        logits = jax.lax.dot_general(
            h1n, soft_ref[...], (((1,), (0,)), ((), ())),
            preferred_element_type=_F32, precision=_HI)
        m = jnp.max(logits)
        logp = logits - (m + jnp.log(jnp.sum(jnp.exp(logits - m))))
        z = logits + gum_ref[pl.ds(t, 1), :]
        a = jnp.min(jnp.where(z >= jnp.max(z), iota_a, _A)).astype(jnp.int32)
        onehot = iota_a == a
        lp = jnp.sum(jnp.where(onehot, logp, 0.0))
        ent = -jnp.sum(jnp.exp(logp) * logp)
        oh9 = (jax.lax.broadcasted_iota(jnp.int32, (1, 9), 1) == a + 1)
        x0e_next = jax.lax.dot_general(
            oh9.astype(_F32), e_ref[...], (((1,), (0,)), ((), ())),
            preferred_element_type=_F32, precision=_HI)
        tmask = iota_t == t
        lp_row = jnp.where(tmask, lp, lp_row)
        ent_row = jnp.where(tmask, ent, ent_row)
        act_row = jnp.where(tmask, a, act_row)
        return (x0e_next, h0n, c0n, h1n, c1n, lp_row, ent_row, act_row)

    zvec = jnp.zeros((1, H), _F32)
    init = (e_ref[0:1, :], zvec, zvec, zvec, zvec,
            jnp.zeros((1, _STEPS), _F32), jnp.zeros((1, _STEPS), _F32),
            jnp.zeros((1, _STEPS), jnp.int32))
    carry = jax.lax.fori_loop(0, _STEPS, step_fn, init)
    _, _, _, _, _, lp_row, ent_row, act_row = carry
    stats_ref[0:1, :] = lp_row
    stats_ref[1:2, :] = ent_row
    arch_ref[...] = act_row


def kernel(g_emb, w_emb, soft_emb, W_ih, W_hh, b_ih, b_hh):
    rows = jnp.concatenate([g_emb, w_emb], axis=0)
    e = pl.pallas_call(
        _prep_body,
        out_shape=jax.ShapeDtypeStruct((9, 4096), _F32),
        compiler_params=pltpu.CompilerParams(
            vmem_limit_bytes=64 * 1024 * 1024),
    )(W_ih[0], rows)

    w0h, w0l = _hilo(W_hh[0].T)
    w0 = jnp.concatenate([w0h, w0l], axis=0)
    w1ih, w1il = _hilo(W_ih[1].T)
    w1hh, w1hl = _hilo(W_hh[1].T)
    w1 = jnp.concatenate([w1ih, w1il, w1hh, w1hl], axis=0)
    b = b_ih + b_hh
    gum = _gumbel_table()

    stats, arch_row = pl.pallas_call(
        _main_body,
        out_shape=[
            jax.ShapeDtypeStruct((2, _STEPS), _F32),
            jax.ShapeDtypeStruct((1, _STEPS), jnp.int32),
        ],
        compiler_params=pltpu.CompilerParams(
            vmem_limit_bytes=100 * 1024 * 1024),
    )(e, w0, w1, b, soft_emb, gum)
    return stats, arch_row[0]


# speedup vs baseline: 2.4598x; 1.0029x over previous
import jax
import jax.numpy as jnp
import numpy as np
from jax.experimental import pallas as pl
from jax.experimental.pallas import tpu as pltpu

_STEPS = 24
_A = 8
_H = 1024
_F32 = jnp.float32
_BF16 = jnp.bfloat16
_HI = jax.lax.Precision.HIGHEST


def _gumbel_table():
    skey = jax.random.key(42)
    rows = [
        jax.random.gumbel(jax.random.fold_in(skey, s), (1, _A), _F32)
        for s in range(_STEPS)
    ]
    return jnp.concatenate(rows, axis=0)


def _hilo(w):
    hi = w.astype(_BF16)
    lo = (w - hi.astype(_F32)).astype(_BF16)
    return hi, lo


def _prep_body(w0_ref, rows_ref, e_ref):
    rows_t = rows_ref[...].T
    et = jax.lax.dot_general(
        w0_ref[...], rows_t, (((1,), (0,)), ((), ())),
        preferred_element_type=_F32, precision=_HI)
    e_ref[...] = et.T


def _main_body(e_ref, w0_ref, w1_ref, b_ref, soft_ref, gum_ref,
               stats_ref, arch_ref):
    H = _H
    iota_a = jax.lax.broadcasted_iota(jnp.int32, (1, _A), 1)
    iota_t = jax.lax.broadcasted_iota(jnp.int32, (1, _STEPS), 1)
    b0 = b_ref[0:1, :]
    b1 = b_ref[1:2, :]
    zrow = jnp.zeros((1, H), _BF16)

    def cell(gates, c):
        i_g = gates[:, 0:H]
        f_g = gates[:, H:2 * H]
        g_g = gates[:, 2 * H:3 * H]
        o_g = gates[:, 3 * H:4 * H]
        c_new = jax.nn.sigmoid(f_g) * c + jax.nn.sigmoid(i_g) * jnp.tanh(g_g)
        h_new = jax.nn.sigmoid(o_g) * jnp.tanh(c_new)
        return h_new, c_new

    def bdot(act, w_ref):
        r = jax.lax.dot_general(
            act, w_ref[...], (((1,), (0,)), ((), ())),
            preferred_element_type=_F32)
        return r[0:1, :] + r[1:2, :]

    def step_fn(t, carry):
        x0e, h0, c0, h1, c1, lp_row, ent_row, act_row = carry
        h0h, h0l = _hilo(h0)
        a0 = jnp.concatenate([
            jnp.concatenate([h0h, h0h], axis=1),
            jnp.concatenate([h0l, zrow], axis=1)], axis=0)
        g0 = x0e + bdot(a0, w0_ref) + b0
        h0n, c0n = cell(g0, c0)
        xh, xl = _hilo(h0n)
        hh, hl = _hilo(h1)
        a1 = jnp.concatenate([
            jnp.concatenate([xh, xh, hh, hh], axis=1),
            jnp.concatenate([xl, zrow, hl, zrow], axis=1)], axis=0)
        g1 = bdot(a1, w1_ref) + b1
        h1n, c1n = cell(g1, c1)
        logits = jax.lax.dot_general(
            h1n, soft_ref[...], (((1,), (0,)), ((), ())),
            preferred_element_type=_F32, precision=_HI)
        m = jnp.max(logits)
        logp = logits - (m + jnp.log(jnp.sum(jnp.exp(logits - m))))
        z = logits + gum_ref[pl.ds(t, 1), :]
        a = jnp.min(jnp.where(z >= jnp.max(z), iota_a, _A)).astype(jnp.int32)
        onehot = iota_a == a
        lp = jnp.sum(jnp.where(onehot, logp, 0.0))
        ent = -jnp.sum(jnp.exp(logp) * logp)
        oh9 = (jax.lax.broadcasted_iota(jnp.int32, (1, 9), 1) == a + 1)
        x0e_next = jax.lax.dot_general(
            oh9.astype(_F32), e_ref[...], (((1,), (0,)), ((), ())),
            preferred_element_type=_F32, precision=_HI)
        tmask = iota_t == t
        lp_row = jnp.where(tmask, lp, lp_row)
        ent_row = jnp.where(tmask, ent, ent_row)
        act_row = jnp.where(tmask, a, act_row)
        return (x0e_next, h0n, c0n, h1n, c1n, lp_row, ent_row, act_row)

    zvec = jnp.zeros((1, H), _F32)
    init = (e_ref[0:1, :], zvec, zvec, zvec, zvec,
            jnp.zeros((1, _STEPS), _F32), jnp.zeros((1, _STEPS), _F32),
            jnp.zeros((1, _STEPS), jnp.int32))
    carry = jax.lax.fori_loop(0, _STEPS, step_fn, init)
    _, _, _, _, _, lp_row, ent_row, act_row = carry
    stats_ref[0:1, :] = lp_row
    stats_ref[1:2, :] = ent_row
    arch_ref[...] = act_row


def kernel(g_emb, w_emb, soft_emb, W_ih, W_hh, b_ih, b_hh):
    rows = jnp.concatenate([g_emb, w_emb], axis=0)
    e = pl.pallas_call(
        _prep_body,
        out_shape=jax.ShapeDtypeStruct((9, 4096), _F32),
        compiler_params=pltpu.CompilerParams(
            vmem_limit_bytes=64 * 1024 * 1024),
    )(W_ih[0], rows)

    w0h, w0l = _hilo(W_hh[0].T)
    w0 = jnp.concatenate([w0h, w0l], axis=0)
    w1ih, w1il = _hilo(W_ih[1].T)
    w1hh, w1hl = _hilo(W_hh[1].T)
    w1 = jnp.concatenate([w1ih, w1il, w1hh, w1hl], axis=0)
    b = b_ih + b_hh
    gum = _gumbel_table()

    stats, arch_row = pl.pallas_call(
        _main_body,
        out_shape=[
            jax.ShapeDtypeStruct((2, _STEPS), _F32),
            jax.ShapeDtypeStruct((1, _STEPS), jnp.int32),
        ],
        compiler_params=pltpu.CompilerParams(
            vmem_limit_bytes=100 * 1024 * 1024,
            allow_input_fusion=[False, True, True, False, False, False]),
    )(e, w0, w1, b, soft_emb, gum)
    return stats, arch_row[0]
